# Initial kernel scaffold; baseline (speedup 1.0000x reference)
import jax
import jax.numpy as jnp
from jax.experimental import pallas as pl


def kernel(updates, mask):
    raise NotImplementedError("write your pallas kernel here")


# speedup vs baseline: 24.7320x; 24.7320x over previous
import jax
import jax.numpy as jnp
from jax import lax
from jax.experimental import pallas as pl
from jax.experimental.pallas import tpu as pltpu
from jax.experimental.pallas import tpu_sc as plsc

B = 4
H = W = 192
C = 96
HW = H * W
P = (2 * H) * (2 * W)
CB = 4
NBLK = C // CB
NC = 2
NS = 16
LANES = 16
POS_PER_TILE = HW // NS
NELEM = POS_PER_TILE * CB
ROWS_PER_TILE = P // NS
ACC_WORDS = CB * P
ZCHUNK = NELEM
NTASK = B * NBLK
VPC = POS_PER_TILE // LANES


def _body(mask_hbm, upd_hbm, out_hbm, mbuf, midx, ubuf, zbuf, acc):
  cid = lax.axis_index("c")
  sid = lax.axis_index("s")

  zeros16 = jnp.zeros((LANES,), jnp.float32)

  def zfill(i, _):
    zbuf[pl.ds(i * LANES, LANES)] = zeros16
    return 0
  lax.fori_loop(0, ZCHUNK // LANES, zfill, 0)

  def zero_acc():
    base = sid * (ACC_WORDS // NS)
    for j in range(ACC_WORDS // NS // ZCHUNK):
      pltpu.sync_copy(zbuf, acc.at[pl.ds(base + j * ZCHUNK, ZCHUNK)])

  zero_acc()
  plsc.subcore_barrier()

  def task(t, _):
    task_id = t * NC + cid
    b = task_id // NBLK
    blk = task_id % NBLK
    pos0 = sid * POS_PER_TILE
    c0 = blk * CB

    for k in range(CB):
      pltpu.sync_copy(
          mask_hbm.at[b, c0 + k, pl.ds(pos0, POS_PER_TILE)],
          mbuf.at[pl.ds(k * POS_PER_TILE, POS_PER_TILE)],
      )
      pltpu.sync_copy(
          upd_hbm.at[b, c0 + k, pl.ds(pos0, POS_PER_TILE)],
          ubuf.at[pl.ds(k * POS_PER_TILE, POS_PER_TILE)],
      )

    cvec = jnp.full((LANES,), C, jnp.int32)
    for k in range(CB):
      kvec = jnp.full((LANES,), k * P, jnp.int32)

      def compute(i, _, kvec=kvec, k=k):
        j = (k * VPC + i) * LANES
        m = mbuf[pl.ds(j, LANES)]
        midx[pl.ds(j, LANES)] = lax.div(m, cvec) + kvec
        return 0
      lax.fori_loop(0, VPC, compute, 0)

    pltpu.sync_copy(ubuf, acc.at[midx], add=True)
    plsc.subcore_barrier()

    r0 = sid * ROWS_PER_TILE
    for k in range(CB):
      pltpu.sync_copy(
          acc.at[pl.ds(k * P + r0, ROWS_PER_TILE)],
          out_hbm.at[b, c0 + k, pl.ds(r0, ROWS_PER_TILE)],
      )
    zero_acc()
    plsc.subcore_barrier()
    return 0

  lax.fori_loop(0, NTASK // NC, task, 0)


@jax.jit
def kernel(updates, mask):
  mask_t = jnp.transpose(mask.astype(jnp.int32).reshape(B, HW, C), (0, 2, 1))
  upd_t = jnp.transpose(updates.reshape(B, HW, C), (0, 2, 1))
  mesh = plsc.VectorSubcoreMesh(
      core_axis_name="c", subcore_axis_name="s", num_cores=NC, num_subcores=NS
  )
  out_t = pl.kernel(
      _body,
      out_type=jax.ShapeDtypeStruct((B, C, P), jnp.float32),
      mesh=mesh,
      scratch_types=[
          pltpu.VMEM((NELEM,), jnp.int32),
          pltpu.VMEM((NELEM,), jnp.int32),
          pltpu.VMEM((NELEM,), jnp.float32),
          pltpu.VMEM((ZCHUNK,), jnp.float32),
          pltpu.VMEM_SHARED((ACC_WORDS,), jnp.float32),
      ],
  )(mask_t, upd_t)
  return jnp.transpose(out_t, (0, 2, 1)).reshape(B, 2 * H, 2 * W, C)
